# Initial kernel scaffold; baseline (speedup 1.0000x reference)
import jax
import jax.numpy as jnp
from jax.experimental import pallas as pl


def kernel(x, time_indices, W_ft, W_time, W_tk):
    raise NotImplementedError("write your pallas kernel here")


# speedup vs baseline: 6.2755x; 6.2755x over previous
import functools

import jax
import jax.numpy as jnp
from jax import lax
from jax.experimental import pallas as pl

FB = 64


def _enc_add_kernel(ti_ref, x_ref, wft_ref, wtime_ref, wtk_ref, o_ref):
    i = pl.program_id(0)
    third = wft_ref.shape[1]
    d_model = 3 * third
    batch = x_ref.shape[1]
    max_time = wtime_ref.shape[0]

    col = lax.broadcasted_iota(jnp.int32, (third, d_model), 1)
    row = lax.broadcasted_iota(jnp.int32, (third, d_model), 0)
    base = col - 3 * row
    p0 = (base == 0).astype(jnp.float32)
    p1 = (base == 1).astype(jnp.float32)
    p2 = (base == 2).astype(jnp.float32)

    ft = wft_ref[pl.ds(i, 1), :]
    tk = wtk_ref[...]
    t_iota = lax.broadcasted_iota(jnp.int32, (batch, max_time), 1)
    onehot = (t_iota == ti_ref[...]).astype(jnp.float32)
    tm = jnp.dot(onehot, wtime_ref[...], preferred_element_type=jnp.float32)

    e_f = (jnp.dot(ft, p0, preferred_element_type=jnp.float32)
           + jnp.dot(tk, p2, preferred_element_type=jnp.float32))
    e_b = jnp.dot(tm, p1, preferred_element_type=jnp.float32)

    o_ref[...] = x_ref[...] + e_f[:, None, :] + e_b[None, :, :]


@jax.jit
def kernel(x, time_indices, W_ft, W_time, W_tk):
    num_features, batch, d_model = x.shape
    feature_types, third = W_ft.shape
    n_tickers = W_tk.shape[0]
    max_time = W_time.shape[0]
    ti = time_indices.astype(jnp.int32).reshape(batch, 1)

    grid = (feature_types, n_tickers // FB)
    return pl.pallas_call(
        _enc_add_kernel,
        grid=grid,
        in_specs=[
            pl.BlockSpec((batch, 1), lambda i, j: (0, 0)),
            pl.BlockSpec((FB, batch, d_model),
                         lambda i, j: (i * (n_tickers // FB) + j, 0, 0)),
            pl.BlockSpec((feature_types, third), lambda i, j: (0, 0)),
            pl.BlockSpec((max_time, third), lambda i, j: (0, 0)),
            pl.BlockSpec((FB, third), lambda i, j: (j, 0)),
        ],
        out_specs=pl.BlockSpec((FB, batch, d_model),
                               lambda i, j: (i * (n_tickers // FB) + j, 0, 0)),
        out_shape=jax.ShapeDtypeStruct(x.shape, x.dtype),
    )(ti, x, W_ft, W_time, W_tk)


# speedup vs baseline: 6.4117x; 1.0217x over previous
import functools

import jax
import jax.numpy as jnp
from jax import lax
from jax.experimental import pallas as pl

FB = 128


def _enc_add_kernel(ti_ref, x_ref, wft_ref, wtime_ref, wtk_ref, o_ref):
    i = pl.program_id(0)
    third = wft_ref.shape[1]
    d_model = 3 * third
    batch = x_ref.shape[1]
    max_time = wtime_ref.shape[0]

    col = lax.broadcasted_iota(jnp.int32, (third, d_model), 1)
    row = lax.broadcasted_iota(jnp.int32, (third, d_model), 0)
    base = col - 3 * row
    p0 = (base == 0).astype(jnp.float32)
    p1 = (base == 1).astype(jnp.float32)
    p2 = (base == 2).astype(jnp.float32)

    ft = wft_ref[pl.ds(i, 1), :]
    tk = wtk_ref[...]
    t_iota = lax.broadcasted_iota(jnp.int32, (batch, max_time), 1)
    onehot = (t_iota == ti_ref[...]).astype(jnp.float32)
    tm = jnp.dot(onehot, wtime_ref[...], preferred_element_type=jnp.float32)

    e_f = (jnp.dot(ft, p0, preferred_element_type=jnp.float32)
           + jnp.dot(tk, p2, preferred_element_type=jnp.float32))
    e_b = jnp.dot(tm, p1, preferred_element_type=jnp.float32)

    o_ref[...] = x_ref[...] + e_f[:, None, :] + e_b[None, :, :]


@jax.jit
def kernel(x, time_indices, W_ft, W_time, W_tk):
    num_features, batch, d_model = x.shape
    feature_types, third = W_ft.shape
    n_tickers = W_tk.shape[0]
    max_time = W_time.shape[0]
    ti = time_indices.astype(jnp.int32).reshape(batch, 1)

    grid = (feature_types, n_tickers // FB)
    return pl.pallas_call(
        _enc_add_kernel,
        grid=grid,
        in_specs=[
            pl.BlockSpec((batch, 1), lambda i, j: (0, 0)),
            pl.BlockSpec((FB, batch, d_model),
                         lambda i, j: (i * (n_tickers // FB) + j, 0, 0)),
            pl.BlockSpec((feature_types, third), lambda i, j: (0, 0)),
            pl.BlockSpec((max_time, third), lambda i, j: (0, 0)),
            pl.BlockSpec((FB, third), lambda i, j: (j, 0)),
        ],
        out_specs=pl.BlockSpec((FB, batch, d_model),
                               lambda i, j: (i * (n_tickers // FB) + j, 0, 0)),
        out_shape=jax.ShapeDtypeStruct(x.shape, x.dtype),
    )(ti, x, W_ft, W_time, W_tk)
